# Initial kernel scaffold; baseline (speedup 1.0000x reference)
import jax
import jax.numpy as jnp
from jax.experimental import pallas as pl


def kernel(x, W, binary_mask, bias):
    raise NotImplementedError("write your pallas kernel here")


# speedup vs baseline: 2.0158x; 2.0158x over previous
import jax
import jax.numpy as jnp
from jax.experimental import pallas as pl

N_TOK = 1024
D_MODEL = 2048
BJ = 256


def _snip_fwd_kernel(x_ref, w_ref, m_ref, b_ref, o_ref):
    w = w_ref[...] * m_ref[...]
    acc = jax.lax.dot_general(
        x_ref[...],
        w,
        dimension_numbers=(((1,), (1,)), ((), ())),
        preferred_element_type=jnp.float32,
    )
    o_ref[...] = acc + b_ref[...]


def kernel(x, W, binary_mask, bias):
    bias2d = bias.reshape(1, D_MODEL)
    grid = (D_MODEL // BJ,)
    return pl.pallas_call(
        _snip_fwd_kernel,
        grid=grid,
        in_specs=[
            pl.BlockSpec((N_TOK, D_MODEL), lambda j: (0, 0)),
            pl.BlockSpec((BJ, D_MODEL), lambda j: (j, 0)),
            pl.BlockSpec((BJ, D_MODEL), lambda j: (j, 0)),
            pl.BlockSpec((1, BJ), lambda j: (0, j)),
        ],
        out_specs=pl.BlockSpec((N_TOK, BJ), lambda j: (0, j)),
        out_shape=jax.ShapeDtypeStruct((N_TOK, D_MODEL), jnp.float32),
    )(x, W, binary_mask, bias2d)


# speedup vs baseline: 2.0159x; 1.0000x over previous
import jax
import jax.numpy as jnp
from jax.experimental import pallas as pl

N_TOK = 1024
D_MODEL = 2048
BJ = 256


def _snip_fwd_kernel(x_ref, w_ref, m_ref, b_ref, o_ref):
    w = w_ref[...] * m_ref[...]
    acc = jax.lax.dot_general(
        x_ref[...],
        w,
        dimension_numbers=(((1,), (1,)), ((), ())),
        preferred_element_type=jnp.float32,
        precision=jax.lax.Precision.DEFAULT,
    )
    o_ref[...] = acc + b_ref[...]


def kernel(x, W, binary_mask, bias):
    bias2d = bias.reshape(1, D_MODEL)
    grid = (D_MODEL // BJ,)
    return pl.pallas_call(
        _snip_fwd_kernel,
        grid=grid,
        in_specs=[
            pl.BlockSpec((N_TOK, D_MODEL), lambda j: (0, 0)),
            pl.BlockSpec((BJ, D_MODEL), lambda j: (j, 0)),
            pl.BlockSpec((BJ, D_MODEL), lambda j: (j, 0)),
            pl.BlockSpec((1, BJ), lambda j: (0, j)),
        ],
        out_specs=pl.BlockSpec((N_TOK, BJ), lambda j: (0, j)),
        out_shape=jax.ShapeDtypeStruct((N_TOK, D_MODEL), jnp.float32),
    )(x, W, binary_mask, bias2d)
